# baseline (speedup 1.0000x reference)
import jax
import jax.numpy as jnp
from jax.experimental import pallas as pl
from jax.experimental.pallas import tpu as pltpu

_N = 524288
_C = 100
_BLOCK = 4096


def _cm_kernel(yt_ref, yp_ref, out_ref, cm_ref):
    i = pl.program_id(0)

    @pl.when(i == 0)
    def _init():
        cm_ref[...] = jnp.zeros_like(cm_ref)

    yt = yt_ref[...]
    yp = yp_ref[...]
    t_max = jnp.max(yt, axis=1, keepdims=True)
    p_max = jnp.max(yp, axis=1, keepdims=True)
    oh_t = (yt == t_max).astype(jnp.float32)
    oh_p = (yp == p_max).astype(jnp.float32)
    cm_ref[...] += jax.lax.dot_general(
        oh_t, oh_p, (((0,), (0,)), ((), ())),
        preferred_element_type=jnp.float32)

    @pl.when(i == pl.num_programs(0) - 1)
    def _fin():
        out_ref[...] = cm_ref[...]


def _spec_kernel(cm_ref, out_ref):
    cm = cm_ref[...]
    r = jax.lax.broadcasted_iota(jnp.int32, (_C, _C), 0)
    c = jax.lax.broadcasted_iota(jnp.int32, (_C, _C), 1)
    tp = jnp.sum(jnp.where(r == c, cm, 0.0), axis=0)
    col = jnp.sum(cm, axis=0)
    row = jnp.sum(cm, axis=1)
    fp = col - tp
    fn = row - tp
    tn = jnp.float32(_N) - (tp + fp + fn)
    eps = jnp.finfo(jnp.float32).eps
    spec = tn / (tn + fp + eps)
    out_ref[0, 0] = jnp.sum(spec) / jnp.float32(_C)


@jax.jit
def kernel(y_true, y_pred):
    grid = _N // _BLOCK
    cm = pl.pallas_call(
        _cm_kernel,
        grid=(grid,),
        in_specs=[
            pl.BlockSpec((_BLOCK, _C), lambda i: (i, 0)),
            pl.BlockSpec((_BLOCK, _C), lambda i: (i, 0)),
        ],
        out_specs=pl.BlockSpec((_C, _C), lambda i: (0, 0)),
        out_shape=jax.ShapeDtypeStruct((_C, _C), jnp.float32),
        scratch_shapes=[pltpu.VMEM((_C, _C), jnp.float32)],
    )(y_true, y_pred)
    out = pl.pallas_call(
        _spec_kernel,
        out_specs=pl.BlockSpec((1, 1), memory_space=pltpu.SMEM),
        out_shape=jax.ShapeDtypeStruct((1, 1), jnp.float32),
    )(cm)
    return out[0, 0]


# speedup vs baseline: 1.0007x; 1.0007x over previous
import jax
import jax.numpy as jnp
from jax.experimental import pallas as pl
from jax.experimental.pallas import tpu as pltpu

_N = 524288
_C = 100
_BLOCK = 4096


def _cm_kernel(yt_ref, yp_ref, out_ref, cm_ref):
    i = pl.program_id(0)

    @pl.when(i == 0)
    def _init():
        cm_ref[...] = jnp.zeros_like(cm_ref)

    yt = yt_ref[...]
    yp = yp_ref[...]
    t_max = jnp.max(yt, axis=1, keepdims=True)
    p_max = jnp.max(yp, axis=1, keepdims=True)
    oh_t = (yt == t_max).astype(jnp.float32)
    oh_p = (yp == p_max).astype(jnp.float32)
    cm_ref[...] += jax.lax.dot_general(
        oh_t, oh_p, (((0,), (0,)), ((), ())),
        preferred_element_type=jnp.float32)

    @pl.when(i == pl.num_programs(0) - 1)
    def _fin():
        out_ref[...] = cm_ref[...]


def _spec_kernel(cm_ref, out_ref):
    cm = cm_ref[...]
    r = jax.lax.broadcasted_iota(jnp.int32, (_C, _C), 0)
    c = jax.lax.broadcasted_iota(jnp.int32, (_C, _C), 1)
    tp = jnp.sum(jnp.where(r == c, cm, 0.0), axis=0)
    col = jnp.sum(cm, axis=0)
    row = jnp.sum(cm, axis=1)
    fp = col - tp
    fn = row - tp
    tn = jnp.float32(_N) - (tp + fp + fn)
    eps = jnp.finfo(jnp.float32).eps
    spec = tn / (tn + fp + eps)
    out_ref[0, 0] = jnp.sum(spec) / jnp.float32(_C)


@jax.jit
def kernel(y_true, y_pred):
    grid = _N // _BLOCK
    cm = pl.pallas_call(
        _cm_kernel,
        grid=(grid,),
        in_specs=[
            pl.BlockSpec((_BLOCK, _C), lambda i: (i, 0)),
            pl.BlockSpec((_BLOCK, _C), lambda i: (i, 0)),
        ],
        out_specs=pl.BlockSpec((_C, _C), lambda i: (0, 0)),
        out_shape=jax.ShapeDtypeStruct((_C, _C), jnp.float32),
        scratch_shapes=[pltpu.VMEM((_C, _C), jnp.float32)],
        compiler_params=pltpu.CompilerParams(skip_device_barrier=True),
    )(y_true, y_pred)
    out = pl.pallas_call(
        _spec_kernel,
        out_specs=pl.BlockSpec((1, 1), memory_space=pltpu.SMEM),
        out_shape=jax.ShapeDtypeStruct((1, 1), jnp.float32),
        compiler_params=pltpu.CompilerParams(skip_device_barrier=True),
    )(cm)
    return out[0, 0]


# speedup vs baseline: 1.0798x; 1.0790x over previous
import jax
import jax.numpy as jnp
from jax import lax
from jax.experimental import pallas as pl
from jax.experimental.pallas import tpu as pltpu
from jax.experimental.pallas import tpu_sc as plsc

_N = 524288
_C = 100
_BLOCK = 4096
_M = 323584
_NW = 32
_RW = (_N - _M) // _NW
_BATCH = 128
_NB = _RW // _BATCH
_NBINS = _C * _C
_OFFS = (0, 16, 32, 48, 64, 80, 84)


def _cm_kernel(yt_ref, yp_ref, out_ref, cm_ref):
    i = pl.program_id(0)

    @pl.when(i == 0)
    def _init():
        cm_ref[...] = jnp.zeros_like(cm_ref)

    yt = yt_ref[...]
    yp = yp_ref[...]
    t_max = jnp.max(yt, axis=1, keepdims=True)
    p_max = jnp.max(yp, axis=1, keepdims=True)
    oh_t = (yt == t_max).astype(jnp.float32)
    oh_p = (yp == p_max).astype(jnp.float32)
    cm_ref[...] += jax.lax.dot_general(
        oh_t, oh_p, (((0,), (0,)), ((), ())),
        preferred_element_type=jnp.float32)

    @pl.when(i == pl.num_programs(0) - 1)
    def _fin():
        out_ref[...] = cm_ref[...]


def _row_argmax(buf, i, idx_consts):
    v = [buf[i, off:off + 16] for off in _OFFS]
    m = v[0]
    for k in range(1, 7):
        m = jnp.maximum(m, v[k])
    ms = lax.reduce_max(m, (0,))
    idx = jnp.full((16,), 200.0, jnp.float32)
    for k in range(7):
        idx = jnp.minimum(idx, jnp.where(v[k] == ms, idx_consts[k], 200.0))
    return lax.reduce_min(idx, (0,))


def _sc_hist(yt_hbm, yp_hbm, out_hbm, tbufs, pbufs, hist, sems):
    c = lax.axis_index("c")
    s = lax.axis_index("s")
    wid = s * 2 + c
    base = _M + wid * _RW

    lane_i = lax.iota(jnp.int32, 16)
    lane_f = lane_i.astype(jnp.float32)
    idx_consts = [lane_f + jnp.float32(off) for off in _OFFS]
    ones = jnp.ones((16,), jnp.float32)

    def _zero_hist(i, _):
        hist[pl.ds(i * 16, 16)] = jnp.zeros((16,), jnp.float32)
        return 0

    lax.fori_loop(0, _NBINS // 16, _zero_hist, 0)

    def _start(j, b):
        pltpu.make_async_copy(
            yt_hbm.at[pl.ds(base + j * _BATCH, _BATCH), :],
            tbufs[b], sems[b]).start()
        pltpu.make_async_copy(
            yp_hbm.at[pl.ds(base + j * _BATCH, _BATCH), :],
            pbufs[b], sems[b]).start()

    def _wait(j, b):
        pltpu.make_async_copy(
            yt_hbm.at[pl.ds(base + j * _BATCH, _BATCH), :],
            tbufs[b], sems[b]).wait()
        pltpu.make_async_copy(
            yp_hbm.at[pl.ds(base + j * _BATCH, _BATCH), :],
            pbufs[b], sems[b]).wait()

    def _consume(b):
        tb = tbufs[b]
        pb = pbufs[b]

        def _group(g, _):
            bins = jnp.zeros((16,), jnp.float32)
            for r in range(16):
                i = g * 16 + r
                t = _row_argmax(tb, i, idx_consts)
                p = _row_argmax(pb, i, idx_consts)
                flat = t * jnp.float32(_C) + p
                bins = jnp.where(lane_i == r, flat, bins)
            plsc.addupdate_scatter(hist, [bins.astype(jnp.int32)], ones)
            return 0

        lax.fori_loop(0, _BATCH // 16, _group, 0)

    _start(0, 0)

    def _outer(h, _):
        j0 = h * 2

        @pl.when(j0 + 1 < _NB)
        def _s1():
            _start(j0 + 1, 1)

        _wait(j0, 0)
        _consume(0)

        @pl.when(j0 + 2 < _NB)
        def _s2():
            _start(j0 + 2, 0)

        @pl.when(j0 + 1 < _NB)
        def _c1():
            _wait(j0 + 1, 1)
            _consume(1)

        return 0

    lax.fori_loop(0, (_NB + 1) // 2, _outer, 0)
    pltpu.sync_copy(hist, out_hbm.at[wid])


def _spec_kernel(cm_ref, hists_ref, out_ref):
    cm = cm_ref[...] + jnp.sum(hists_ref[...], axis=0)
    r = jax.lax.broadcasted_iota(jnp.int32, (_C, _C), 0)
    c = jax.lax.broadcasted_iota(jnp.int32, (_C, _C), 1)
    tp = jnp.sum(jnp.where(r == c, cm, 0.0), axis=0)
    col = jnp.sum(cm, axis=0)
    row = jnp.sum(cm, axis=1)
    fp = col - tp
    fn = row - tp
    tn = jnp.float32(_N) - (tp + fp + fn)
    eps = jnp.finfo(jnp.float32).eps
    spec = tn / (tn + fp + eps)
    out_ref[0, 0] = jnp.sum(spec) / jnp.float32(_C)


@jax.jit
def kernel(y_true, y_pred):
    sc_hists = pl.kernel(
        _sc_hist,
        out_type=jax.ShapeDtypeStruct((_NW, _NBINS), jnp.float32),
        mesh=plsc.VectorSubcoreMesh(core_axis_name="c", subcore_axis_name="s"),
        scratch_types=[
            [pltpu.VMEM((_BATCH, _C), jnp.float32) for _ in range(2)],
            [pltpu.VMEM((_BATCH, _C), jnp.float32) for _ in range(2)],
            pltpu.VMEM((_NBINS,), jnp.float32),
            [pltpu.SemaphoreType.DMA for _ in range(2)],
        ],
        compiler_params=pltpu.CompilerParams(
            use_tc_tiling_on_sc=True, needs_layout_passes=False),
    )(y_true, y_pred)

    grid = _M // _BLOCK
    cm = pl.pallas_call(
        _cm_kernel,
        grid=(grid,),
        in_specs=[
            pl.BlockSpec((_BLOCK, _C), lambda i: (i, 0)),
            pl.BlockSpec((_BLOCK, _C), lambda i: (i, 0)),
        ],
        out_specs=pl.BlockSpec((_C, _C), lambda i: (0, 0)),
        out_shape=jax.ShapeDtypeStruct((_C, _C), jnp.float32),
        scratch_shapes=[pltpu.VMEM((_C, _C), jnp.float32)],
        compiler_params=pltpu.CompilerParams(skip_device_barrier=True),
    )(y_true, y_pred)

    hists3 = sc_hists.reshape(_NW, _C, _C)
    out = pl.pallas_call(
        _spec_kernel,
        out_specs=pl.BlockSpec((1, 1), memory_space=pltpu.SMEM),
        out_shape=jax.ShapeDtypeStruct((1, 1), jnp.float32),
        compiler_params=pltpu.CompilerParams(skip_device_barrier=True),
    )(cm, hists3)
    return out[0, 0]
